# baseline (speedup 1.0000x reference)
import functools
import math

import jax
import jax.numpy as jnp
import numpy as np
from jax import lax
from jax.experimental import pallas as pl
from jax.experimental.pallas import tpu as pltpu
from jax.experimental.pallas import tpu_sc as plsc

_PAD = 0
_SMOOTHING = 0.1
_CONF = 1.0 - _SMOOTHING


def _tc_body(smooth, K, x_ref, t_ref, out_ref):
    i = pl.program_id(0)
    xb = x_ref[...]
    tb = t_ref[0, 0, :]
    rowsum = jnp.sum(xb, axis=1)
    col0 = xb[:, 0]
    valid = tb != _PAD
    contrib = jnp.where(valid, K - smooth * (rowsum - col0), 0.0)
    s = jnp.sum(contrib).reshape(1, 1)

    @pl.when(i == 0)
    def _():
        out_ref[...] = jnp.zeros_like(out_ref)

    out_ref[...] += s


def _tc_partial(x, target, smooth, K):
    n, size = x.shape
    BR = 64
    nb = n // BR
    t3 = target.reshape(nb, 1, BR)
    out = pl.pallas_call(
        functools.partial(_tc_body, smooth, K),
        grid=(nb,),
        in_specs=[
            pl.BlockSpec((BR, size), lambda i: (i, 0)),
            pl.BlockSpec((1, 1, BR), lambda i: (i, 0, 0)),
        ],
        out_specs=pl.BlockSpec((1, 1), lambda i: (0, 0)),
        out_shape=jax.ShapeDtypeStruct((1, 1), jnp.float32),
    )(x, t3)
    return out[0, 0]


def _sc_gather_partials(x, target, coef):
    n, size = x.shape
    info = plsc.get_sparse_core_info()
    NC, NS, L = info.num_cores, info.num_subcores, info.num_lanes
    NW = NC * NS
    rpw = n // NW
    nblk = size // 128
    x2 = x.reshape(n * nblk, 128)
    coef = float(coef)

    def body(x2_hbm, t_hbm, out_hbm, tv, idx_v, rows_v, sbuf, sem):
        wid = lax.axis_index("s") * NC + lax.axis_index("c")
        base = wid * rpw
        pltpu.sync_copy(t_hbm.at[pl.ds(base, rpw)], tv)
        for j in range(rpw // L):
            tb = tv[pl.ds(j * L, L)]
            grow = (base + j * L) + lax.iota(jnp.int32, L)
            idx_v[pl.ds(j * L, L)] = grow * nblk + (tb >> 7)
        pltpu.async_copy(x2_hbm.at[idx_v], rows_v, sem).wait()
        s = jnp.zeros((L,), jnp.float32)
        for j in range(rpw // L):
            tb = tv[pl.ds(j * L, L)]
            rowids = j * L + lax.iota(jnp.int32, L)
            vals = plsc.load_gather(rows_v, [rowids, tb & 127])
            s = s + jnp.where(tb != _PAD, vals * coef, 0.0)
        sbuf[...] = s
        pltpu.sync_copy(sbuf, out_hbm.at[wid])

    mesh = plsc.VectorSubcoreMesh(core_axis_name="c", subcore_axis_name="s")
    run = pl.kernel(
        body,
        mesh=mesh,
        compiler_params=pltpu.CompilerParams(needs_layout_passes=False),
        out_type=jax.ShapeDtypeStruct((NW, L), jnp.float32),
        scratch_types=[
            pltpu.VMEM((rpw,), jnp.int32),
            pltpu.VMEM((rpw,), jnp.int32),
            pltpu.VMEM((rpw, 128), jnp.float32),
            pltpu.VMEM((L,), jnp.float32),
            pltpu.SemaphoreType.DMA,
        ],
    )
    return run(x2, target)


def kernel(x, target):
    n, size = x.shape
    smooth = float(np.float32(_SMOOTHING / (size - 2)))
    K = (size - 2) * smooth * math.log(smooth) + _CONF * math.log(_CONF)

    sA = _tc_partial(x, target, smooth, K)
    sB = _sc_gather_partials(x, target, smooth - _CONF)
    return sA + jnp.sum(sB)


# speedup vs baseline: 2.7357x; 2.7357x over previous
import functools
import math

import jax
import jax.numpy as jnp
import numpy as np
from jax import lax
from jax.experimental import pallas as pl
from jax.experimental.pallas import tpu as pltpu
from jax.experimental.pallas import tpu_sc as plsc

_PAD = 0
_SMOOTHING = 0.1
_CONF = 1.0 - _SMOOTHING


def _tc_body(smooth, K, x_ref, t_ref, out_ref):
    i = pl.program_id(0)
    xb = x_ref[...]
    tb = t_ref[0, 0, :]
    rowsum = jnp.sum(xb, axis=1)
    col0 = xb[:, 0]
    valid = tb != _PAD
    contrib = jnp.where(valid, K - smooth * (rowsum - col0), 0.0)
    s = jnp.sum(contrib).reshape(1, 1)

    @pl.when(i == 0)
    def _():
        out_ref[...] = jnp.zeros_like(out_ref)

    out_ref[...] += s


def _tc_partial(x, target, smooth, K):
    n, size = x.shape
    BR = 64
    nb = n // BR
    t3 = target.reshape(nb, 1, BR)
    out = pl.pallas_call(
        functools.partial(_tc_body, smooth, K),
        grid=(nb,),
        in_specs=[
            pl.BlockSpec((BR, size), lambda i: (i, 0)),
            pl.BlockSpec((1, 1, BR), lambda i: (i, 0, 0)),
        ],
        out_specs=pl.BlockSpec((1, 1), lambda i: (0, 0)),
        out_shape=jax.ShapeDtypeStruct((1, 1), jnp.float32),
    )(x, t3)
    return out[0, 0]


def _sc_gather_partials(x, target, coef):
    n, size = x.shape
    info = plsc.get_sparse_core_info()
    NC, NS, L = info.num_cores, info.num_subcores, info.num_lanes
    NW = NC * NS
    rpw = n // NW
    coef = float(coef)

    def body(x_hbm, t_hbm, out_hbm, tv, rowbuf, sbuf, sem):
        wid = lax.axis_index("s") * NC + lax.axis_index("c")
        base = wid * rpw
        pltpu.sync_copy(t_hbm.at[pl.ds(base, rpw)], tv)

        for j in range(rpw // L):
            tvj = tv[pl.ds(j * L, L)]
            c0j = tvj & -128
            for l in range(L):
                i = j * L + l
                pltpu.async_copy(
                    x_hbm.at[pl.ds(base + (i & -8), 8),
                             pl.ds(pl.multiple_of(c0j[l], 128), 128)],
                    rowbuf.at[i], sem)

        def drain(i, carry):
            pltpu.make_async_copy(
                x_hbm.at[pl.ds(0, 8), pl.ds(0, 128)], rowbuf.at[i], sem).wait()
            return carry

        lax.fori_loop(0, rpw, drain, 0)

        s = jnp.zeros((L,), jnp.float32)
        for j in range(rpw // L):
            tb = tv[pl.ds(j * L, L)]
            rowids = j * L + lax.iota(jnp.int32, L)
            vals = plsc.load_gather(rowbuf, [rowids, rowids & 7, tb & 127])
            s = s + jnp.where(tb != _PAD, vals * coef, 0.0)
        sbuf[...] = s
        pltpu.sync_copy(sbuf, out_hbm.at[wid])

    mesh = plsc.VectorSubcoreMesh(core_axis_name="c", subcore_axis_name="s")
    run = pl.kernel(
        body,
        mesh=mesh,
        compiler_params=pltpu.CompilerParams(needs_layout_passes=False),
        out_type=jax.ShapeDtypeStruct((NW, L), jnp.float32),
        scratch_types=[
            pltpu.VMEM((rpw,), jnp.int32),
            pltpu.VMEM((rpw, 8, 128), jnp.float32),
            pltpu.VMEM((L,), jnp.float32),
            pltpu.SemaphoreType.DMA,
        ],
    )
    return run(x, target)


def kernel(x, target):
    n, size = x.shape
    smooth = float(np.float32(_SMOOTHING / (size - 2)))
    K = (size - 2) * smooth * math.log(smooth) + _CONF * math.log(_CONF)

    sA = _tc_partial(x, target, smooth, K)
    sB = _sc_gather_partials(x, target, smooth - _CONF)
    return sA + jnp.sum(sB)


# speedup vs baseline: 2.7391x; 1.0013x over previous
import functools
import math

import jax
import jax.numpy as jnp
import numpy as np
from jax import lax
from jax.experimental import pallas as pl
from jax.experimental.pallas import tpu as pltpu
from jax.experimental.pallas import tpu_sc as plsc

_PAD = 0
_SMOOTHING = 0.1
_CONF = 1.0 - _SMOOTHING


def _tc_body(smooth, K, x_ref, t_ref, out_ref):
    i = pl.program_id(0)
    xb = x_ref[...]
    tb = t_ref[0, 0, :]
    rowsum = jnp.sum(xb, axis=1)
    col0 = xb[:, 0]
    valid = tb != _PAD
    contrib = jnp.where(valid, K - smooth * (rowsum - col0), 0.0)
    s = jnp.sum(contrib).reshape(1, 1)

    @pl.when(i == 0)
    def _():
        out_ref[...] = jnp.zeros_like(out_ref)

    out_ref[...] += s


def _tc_partial(x, target, smooth, K):
    n, size = x.shape
    BR = 64
    nb = n // BR
    t3 = target.reshape(nb, 1, BR)
    out = pl.pallas_call(
        functools.partial(_tc_body, smooth, K),
        grid=(nb,),
        in_specs=[
            pl.BlockSpec((BR, size), lambda i: (i, 0)),
            pl.BlockSpec((1, 1, BR), lambda i: (i, 0, 0)),
        ],
        out_specs=pl.BlockSpec((1, 1), lambda i: (0, 0)),
        out_shape=jax.ShapeDtypeStruct((1, 1), jnp.float32),
    )(x, t3)
    return out[0, 0]


def _sc_gather_partials(x, target, coef):
    n, size = x.shape
    info = plsc.get_sparse_core_info()
    NC, NS, L = info.num_cores, info.num_subcores, info.num_lanes
    NW = NC * NS
    rpw = n // NW
    coef = float(coef)

    def body(x_hbm, t_hbm, out_hbm, tv, rowbuf, sbuf, sem):
        wid = lax.axis_index("s") * NC + lax.axis_index("c")
        base = wid * rpw
        pltpu.sync_copy(t_hbm.at[pl.ds(base, rpw)], tv)

        for j in range(rpw // L):
            tvj = tv[pl.ds(j * L, L)]
            c0j = tvj & -128
            for l in range(L):
                i = j * L + l
                pltpu.async_copy(
                    x_hbm.at[pl.ds(base + (i & -8), 8),
                             pl.ds(pl.multiple_of(c0j[l], 128), 128)],
                    rowbuf.at[i], sem)

        def drain(i, carry):
            pltpu.make_async_copy(
                x_hbm.at[pl.ds(0, 8), pl.ds(0, 128)], rowbuf.at[i], sem).wait()
            return carry

        lax.fori_loop(0, rpw, drain, 0)

        s = jnp.zeros((L,), jnp.float32)
        for j in range(rpw // L):
            tb = tv[pl.ds(j * L, L)]
            rowids = j * L + lax.iota(jnp.int32, L)
            vals = plsc.load_gather(rowbuf, [rowids, rowids & 7, tb & 127])
            s = s + jnp.where(tb != _PAD, vals * coef, 0.0)
        sbuf[...] = s
        pltpu.sync_copy(sbuf, out_hbm.at[wid])

    mesh = plsc.VectorSubcoreMesh(core_axis_name="c", subcore_axis_name="s")
    run = pl.kernel(
        body,
        mesh=mesh,
        compiler_params=pltpu.CompilerParams(needs_layout_passes=False),
        out_type=jax.ShapeDtypeStruct((NW, L), jnp.float32),
        scratch_types=[
            pltpu.VMEM((rpw,), jnp.int32),
            pltpu.VMEM((rpw, 8, 128), jnp.float32),
            pltpu.VMEM((L,), jnp.float32),
            pltpu.SemaphoreType.DMA,
        ],
    )
    return run(x, target)


def kernel(x, target):
    n, size = x.shape
    smooth = float(np.float32(_SMOOTHING / (size - 2)))
    K = (size - 2) * smooth * math.log(smooth) + _CONF * math.log(_CONF)

    sB = _sc_gather_partials(x, target, smooth - _CONF)
    sA = _tc_partial(x, target, smooth, K)
    return sA + jnp.sum(sB)


# speedup vs baseline: 2.7395x; 1.0001x over previous
import functools
import math

import jax
import jax.numpy as jnp
import numpy as np
from jax import lax
from jax.experimental import pallas as pl
from jax.experimental.pallas import tpu as pltpu
from jax.experimental.pallas import tpu_sc as plsc

_PAD = 0
_SMOOTHING = 0.1
_CONF = 1.0 - _SMOOTHING


def _tc_body(smooth, K, x_ref, t_ref, out_ref):
    i = pl.program_id(0)
    xb = x_ref[...]
    tb = t_ref[0, 0, :]
    rowsum = jnp.sum(xb, axis=1)
    col0 = xb[:, 0]
    valid = tb != _PAD
    contrib = jnp.where(valid, K - smooth * (rowsum - col0), 0.0)
    s = jnp.sum(contrib).reshape(1, 1)

    @pl.when(i == 0)
    def _():
        out_ref[...] = jnp.zeros_like(out_ref)

    out_ref[...] += s


def _tc_partial(x, target, smooth, K):
    n, size = x.shape
    BR = 64
    nb = n // BR
    t3 = target.reshape(nb, 1, BR)
    out = pl.pallas_call(
        functools.partial(_tc_body, smooth, K),
        grid=(nb,),
        in_specs=[
            pl.BlockSpec((BR, size), lambda i: (i, 0)),
            pl.BlockSpec((1, 1, BR), lambda i: (i, 0, 0)),
        ],
        out_specs=pl.BlockSpec((1, 1), lambda i: (0, 0)),
        out_shape=jax.ShapeDtypeStruct((1, 1), jnp.float32),
    )(x, t3)
    return out[0, 0]


def _sc_gather_partials(x, target, coef):
    n, size = x.shape
    info = plsc.get_sparse_core_info()
    NC, NS, L = info.num_cores, info.num_subcores, info.num_lanes
    NW = NC * NS
    rpw = n // NW
    coef = float(coef)

    def body(x_hbm, t_hbm, out_hbm, tv, rowbuf, sbuf, sem):
        wid = lax.axis_index("s") * NC + lax.axis_index("c")
        base = wid * rpw
        pltpu.sync_copy(t_hbm.at[pl.ds(base, rpw)], tv)

        for j in range(rpw // L):
            tvj = tv[pl.ds(j * L, L)]
            c0j = tvj & -128
            for l in range(L):
                i = j * L + l
                pltpu.async_copy(
                    x_hbm.at[pl.ds(base + (i & -8), 8),
                             pl.ds(pl.multiple_of(c0j[l], 128), 128)],
                    rowbuf.at[i], sem)

        def drain(i, carry):
            pltpu.make_async_copy(
                x_hbm.at[pl.ds(0, 8), pl.ds(0, 128)], rowbuf.at[i], sem).wait()
            return carry

        lax.fori_loop(0, rpw, drain, 0)

        s = jnp.zeros((L,), jnp.float32)
        for j in range(rpw // L):
            tb = tv[pl.ds(j * L, L)]
            rowids = j * L + lax.iota(jnp.int32, L)
            vals = plsc.load_gather(rowbuf, [rowids, rowids & 7, tb & 127])
            s = s + jnp.where(tb != _PAD, vals * coef, 0.0)
        sbuf[...] = s
        pltpu.sync_copy(sbuf, out_hbm.at[wid])

    mesh = plsc.VectorSubcoreMesh(core_axis_name="c", subcore_axis_name="s")
    run = pl.kernel(
        body,
        mesh=mesh,
        compiler_params=pltpu.CompilerParams(needs_layout_passes=False),
        cost_estimate=pl.CostEstimate(
            flops=4 * n, transcendentals=0,
            bytes_accessed=n * 8 * 128 * 4 + n * 4 + NW * L * 4),
        out_type=jax.ShapeDtypeStruct((NW, L), jnp.float32),
        scratch_types=[
            pltpu.VMEM((rpw,), jnp.int32),
            pltpu.VMEM((rpw, 8, 128), jnp.float32),
            pltpu.VMEM((L,), jnp.float32),
            pltpu.SemaphoreType.DMA,
        ],
    )
    return run(x, target)


def kernel(x, target):
    n, size = x.shape
    smooth = float(np.float32(_SMOOTHING / (size - 2)))
    K = (size - 2) * smooth * math.log(smooth) + _CONF * math.log(_CONF)

    sB = _sc_gather_partials(x, target, smooth - _CONF)
    sA = _tc_partial(x, target, smooth, K)
    return sA + jnp.sum(sB)


# speedup vs baseline: 2.7752x; 1.0131x over previous
import functools
import math

import jax
import jax.numpy as jnp
import numpy as np
from jax import lax
from jax.experimental import pallas as pl
from jax.experimental.pallas import tpu as pltpu
from jax.experimental.pallas import tpu_sc as plsc

_PAD = 0
_SMOOTHING = 0.1
_CONF = 1.0 - _SMOOTHING


def _tc_body(smooth, K, x_ref, t_ref, out_ref):
    i = pl.program_id(0)
    xb = x_ref[...]
    tb = t_ref[0, 0, :]
    rowsum = jnp.sum(xb, axis=1)
    col0 = xb[:, 0]
    valid = tb != _PAD
    contrib = jnp.where(valid, K - smooth * (rowsum - col0), 0.0)
    s = jnp.sum(contrib).reshape(1, 1)

    @pl.when(i == 0)
    def _():
        out_ref[...] = jnp.zeros_like(out_ref)

    out_ref[...] += s


def _tc_partial(x, target, smooth, K):
    n, size = x.shape
    BR = 128
    nb = n // BR
    t3 = target.reshape(nb, 1, BR)
    out = pl.pallas_call(
        functools.partial(_tc_body, smooth, K),
        grid=(nb,),
        in_specs=[
            pl.BlockSpec((BR, size), lambda i: (i, 0)),
            pl.BlockSpec((1, 1, BR), lambda i: (i, 0, 0)),
        ],
        out_specs=pl.BlockSpec((1, 1), lambda i: (0, 0)),
        out_shape=jax.ShapeDtypeStruct((1, 1), jnp.float32),
    )(x, t3)
    return out[0, 0]


def _sc_gather_partials(x, target, coef):
    n, size = x.shape
    info = plsc.get_sparse_core_info()
    NC, NS, L = info.num_cores, info.num_subcores, info.num_lanes
    NW = NC * NS
    rpw = n // NW
    coef = float(coef)

    def body(x_hbm, t_hbm, out_hbm, tv, rowbuf, sbuf, sem):
        wid = lax.axis_index("s") * NC + lax.axis_index("c")
        base = wid * rpw
        pltpu.sync_copy(t_hbm.at[pl.ds(base, rpw)], tv)

        for j in range(rpw // L):
            tvj = tv[pl.ds(j * L, L)]
            c0j = tvj & -128
            for l in range(L):
                i = j * L + l
                pltpu.async_copy(
                    x_hbm.at[pl.ds(base + (i & -8), 8),
                             pl.ds(pl.multiple_of(c0j[l], 128), 128)],
                    rowbuf.at[i], sem)

        def drain(i, carry):
            pltpu.make_async_copy(
                x_hbm.at[pl.ds(0, 8), pl.ds(0, 128)], rowbuf.at[i], sem).wait()
            return carry

        lax.fori_loop(0, rpw, drain, 0)

        s = jnp.zeros((L,), jnp.float32)
        for j in range(rpw // L):
            tb = tv[pl.ds(j * L, L)]
            rowids = j * L + lax.iota(jnp.int32, L)
            vals = plsc.load_gather(rowbuf, [rowids, rowids & 7, tb & 127])
            s = s + jnp.where(tb != _PAD, vals * coef, 0.0)
        sbuf[...] = s
        pltpu.sync_copy(sbuf, out_hbm.at[wid])

    mesh = plsc.VectorSubcoreMesh(core_axis_name="c", subcore_axis_name="s")
    run = pl.kernel(
        body,
        mesh=mesh,
        compiler_params=pltpu.CompilerParams(needs_layout_passes=False),
        cost_estimate=pl.CostEstimate(
            flops=4 * n, transcendentals=0,
            bytes_accessed=n * 8 * 128 * 4 + n * 4 + NW * L * 4),
        out_type=jax.ShapeDtypeStruct((NW, L), jnp.float32),
        scratch_types=[
            pltpu.VMEM((rpw,), jnp.int32),
            pltpu.VMEM((rpw, 8, 128), jnp.float32),
            pltpu.VMEM((L,), jnp.float32),
            pltpu.SemaphoreType.DMA,
        ],
    )
    return run(x, target)


def kernel(x, target):
    n, size = x.shape
    smooth = float(np.float32(_SMOOTHING / (size - 2)))
    K = (size - 2) * smooth * math.log(smooth) + _CONF * math.log(_CONF)

    sB = _sc_gather_partials(x, target, smooth - _CONF)
    sA = _tc_partial(x, target, smooth, K)
    return sA + jnp.sum(sB)
